# baseline (speedup 1.0000x reference)
import functools

import jax
import jax.numpy as jnp
from jax import lax
from jax.experimental import pallas as pl
from jax.experimental.pallas import tpu as pltpu
from jax.experimental.pallas import tpu_sc as plsc

_NUM_CORES = 2
_NUM_SUBCORES = 16
_NUM_WORKERS = _NUM_CORES * _NUM_SUBCORES


@functools.lru_cache(maxsize=None)
def _make_gather(B, D, chunk):
    b_per_w = B // _NUM_WORKERS
    n_chunks = b_per_w // chunk
    mesh = plsc.VectorSubcoreMesh(core_axis_name="c", subcore_axis_name="s")

    @functools.partial(
        pl.kernel,
        mesh=mesh,
        compiler_params=pltpu.CompilerParams(use_tc_tiling_on_sc=False),
        out_type=jax.ShapeDtypeStruct((B, D), jnp.float32),
        scratch_types=[
            pltpu.VMEM((chunk,), jnp.int32),
            pltpu.VMEM((chunk, D), jnp.float32),
            pltpu.SemaphoreType.DMA,
        ],
    )
    def k(idx_hbm, table_hbm, out_hbm, idx_v, rows_v, sem):
        wid = lax.axis_index("s") * _NUM_CORES + lax.axis_index("c")
        base = wid * b_per_w

        def body(i, carry):
            off = base + i * chunk
            pltpu.sync_copy(idx_hbm.at[pl.ds(off, chunk)], idx_v)
            pltpu.async_copy(table_hbm.at[idx_v], rows_v, sem).wait()
            pltpu.sync_copy(rows_v, out_hbm.at[pl.ds(off, chunk)])
            return carry

        lax.fori_loop(0, n_chunks, body, 0)

    return k


def kernel(token_ids, E):
    batch, fields = token_ids.shape
    B = batch * fields
    D = E.shape[1]
    flat = token_ids.reshape(B).astype(jnp.int32)
    out = _make_gather(B, D, 1024)(flat, E)
    return out.reshape(batch, fields, D)


# speedup vs baseline: 1.0058x; 1.0058x over previous
import functools

import jax
import jax.numpy as jnp
from jax import lax
from jax.experimental import pallas as pl
from jax.experimental.pallas import tpu as pltpu
from jax.experimental.pallas import tpu_sc as plsc

_NUM_CORES = 2
_NUM_SUBCORES = 16
_NUM_WORKERS = _NUM_CORES * _NUM_SUBCORES
_NBUF = 2


@functools.lru_cache(maxsize=None)
def _make_gather(B, D, chunk):
    b_per_w = B // _NUM_WORKERS
    n_chunks = b_per_w // chunk
    assert n_chunks * chunk == b_per_w and n_chunks >= 2
    mesh = plsc.VectorSubcoreMesh(core_axis_name="c", subcore_axis_name="s")

    @functools.partial(
        pl.kernel,
        mesh=mesh,
        compiler_params=pltpu.CompilerParams(use_tc_tiling_on_sc=False),
        out_type=jax.ShapeDtypeStruct((B, D), jnp.float32),
        scratch_types=[
            pltpu.VMEM((_NBUF, chunk), jnp.int32),
            pltpu.VMEM((_NBUF, chunk, D), jnp.float32),
        ]
        + [pltpu.SemaphoreType.DMA] * (3 * _NBUF),
    )
    def k(idx_hbm, table_hbm, out_hbm, idx_v, rows_v, *sems):
        sem_i, sem_g, sem_s = sems[:_NBUF], sems[_NBUF:2 * _NBUF], sems[2 * _NBUF:]
        wid = lax.axis_index("s") * _NUM_CORES + lax.axis_index("c")
        base = wid * b_per_w

        def start_idx(g):
            b = g % _NBUF
            return pltpu.async_copy(
                idx_hbm.at[pl.ds(base + g * chunk, chunk)], idx_v.at[b], sem_i[b])

        def start_gather(g):
            b = g % _NBUF
            return pltpu.async_copy(table_hbm.at[idx_v.at[b]], rows_v.at[b], sem_g[b])

        def start_store(g):
            b = g % _NBUF
            return pltpu.async_copy(
                rows_v.at[b], out_hbm.at[pl.ds(base + g * chunk, chunk)], sem_s[b])

        hi, hg, hs = {}, {}, {}
        hi[0] = start_idx(0)
        hi[1] = start_idx(1)
        hi[0].wait()
        hg[0] = start_gather(0)
        for g in range(1, n_chunks):
            hi[g].wait()
            if g >= _NBUF:
                hs[g - _NBUF].wait()
            hg[g] = start_gather(g)
            hg[g - 1].wait()
            hs[g - 1] = start_store(g - 1)
            if g + 1 < n_chunks:
                hi[g + 1] = start_idx(g + 1)
        hg[n_chunks - 1].wait()
        hs[n_chunks - 1] = start_store(n_chunks - 1)
        hs[n_chunks - 2].wait()
        hs[n_chunks - 1].wait()

    return k


def kernel(token_ids, E):
    batch, fields = token_ids.shape
    B = batch * fields
    D = E.shape[1]
    flat = token_ids.reshape(B).astype(jnp.int32)
    out = _make_gather(B, D, 512)(flat, E)
    return out.reshape(batch, fields, D)


# speedup vs baseline: 1.1458x; 1.1392x over previous
import functools

import jax
import jax.numpy as jnp
from jax import lax
from jax.experimental import pallas as pl
from jax.experimental.pallas import tpu as pltpu
from jax.experimental.pallas import tpu_sc as plsc

_NUM_CORES = 2
_NUM_SUBCORES = 16
_NUM_WORKERS = _NUM_CORES * _NUM_SUBCORES


@functools.lru_cache(maxsize=None)
def _make_gather(B, D, chunk):
    b_per_w = B // _NUM_WORKERS
    n_chunks = b_per_w // chunk
    n_pairs = n_chunks // 2
    assert n_chunks * chunk == b_per_w and n_chunks % 2 == 0 and n_chunks >= 6
    mesh = plsc.VectorSubcoreMesh(core_axis_name="c", subcore_axis_name="s")

    @functools.partial(
        pl.kernel,
        mesh=mesh,
        out_type=jax.ShapeDtypeStruct((B, D), jnp.float32),
        scratch_types=[
            pltpu.VMEM((2, chunk), jnp.int32),
            pltpu.VMEM((2, chunk, D), jnp.float32),
        ]
        + [pltpu.SemaphoreType.DMA] * 6,
    )
    def k(idx_hbm, table_hbm, out_hbm, idx_v, rows_v, *sems):
        sem_i, sem_g, sem_s = sems[:2], sems[2:4], sems[4:]
        wid = lax.axis_index("s") * _NUM_CORES + lax.axis_index("c")
        base = wid * b_per_w

        def issue_idx(g, b):
            pltpu.async_copy(
                idx_hbm.at[pl.ds(base + g * chunk, chunk)], idx_v.at[b], sem_i[b])

        def drain_idx(b):
            pltpu.make_async_copy(
                idx_hbm.at[pl.ds(0, chunk)], idx_v.at[b], sem_i[b]).wait()

        def fire_rows(b):
            for j in range(chunk):
                if j % 16 == 0:
                    vec = idx_v[b, pl.ds(j, 16)]
                tid = vec[j % 16]
                pltpu.async_copy(
                    table_hbm.at[pl.ds(tid, 1)], rows_v.at[b, pl.ds(j, 1)],
                    sem_g[b])

        def drain_rows(b):
            pltpu.make_async_copy(
                table_hbm.at[pl.ds(0, chunk)], rows_v.at[b], sem_g[b]).wait()

        def issue_store(g, b):
            pltpu.async_copy(
                rows_v.at[b], out_hbm.at[pl.ds(base + g * chunk, chunk)], sem_s[b])

        def drain_store(b):
            pltpu.make_async_copy(
                rows_v.at[b], out_hbm.at[pl.ds(0, chunk)], sem_s[b]).wait()

        issue_idx(0, 0)
        issue_idx(1, 1)
        drain_idx(0)
        fire_rows(0)
        issue_idx(2, 0)
        drain_idx(1)
        fire_rows(1)
        issue_idx(3, 1)
        drain_rows(0)
        issue_store(0, 0)

        def body(p, carry):
            for b in (0, 1):
                g = 2 * p + b
                drain_idx(b)
                drain_store(b)
                fire_rows(b)
                issue_idx(g + 2, b)
                o = 1 - b
                drain_rows(o)
                issue_store(g - 1, o)
            return carry

        lax.fori_loop(1, n_pairs - 1, body, 0)

        for b in (0, 1):
            g = n_chunks - 2 + b
            drain_idx(b)
            drain_store(b)
            fire_rows(b)
            o = 1 - b
            drain_rows(o)
            issue_store(g - 1, o)
        drain_rows(1)
        issue_store(n_chunks - 1, 1)
        drain_store(0)
        drain_store(1)

    return k


def kernel(token_ids, E):
    batch, fields = token_ids.shape
    B = batch * fields
    D = E.shape[1]
    flat = token_ids.reshape(B).astype(jnp.int32)
    out = _make_gather(B, D, 128)(flat, E)
    return out.reshape(batch, fields, D)


# speedup vs baseline: 1.3509x; 1.1790x over previous
import functools

import jax
import jax.numpy as jnp
from jax import lax
from jax.experimental import pallas as pl
from jax.experimental.pallas import tpu as pltpu
from jax.experimental.pallas import tpu_sc as plsc

_NUM_CORES = 2
_NUM_SUBCORES = 16
_NUM_WORKERS = _NUM_CORES * _NUM_SUBCORES
_GB = 4


@functools.lru_cache(maxsize=None)
def _make_gather(batch, fields, D):
    b_per_w = batch // _NUM_WORKERS
    n_groups = b_per_w // _GB
    n_pairs = n_groups // 2
    rows = _GB * fields
    assert b_per_w * _NUM_WORKERS == batch and n_groups % 2 == 0 and n_groups >= 6
    assert rows % 8 == 0
    mesh = plsc.VectorSubcoreMesh(core_axis_name="c", subcore_axis_name="s")

    @functools.partial(
        pl.kernel,
        mesh=mesh,
        out_type=jax.ShapeDtypeStruct((batch, fields, D), jnp.float32),
        scratch_types=[
            pltpu.VMEM((2, rows + 16), jnp.int32),
            pltpu.VMEM((2, _GB, fields, D), jnp.float32),
        ]
        + [pltpu.SemaphoreType.DMA] * 6,
    )
    def k(idx_hbm, table_hbm, out_hbm, idx_v, stage_v, *sems):
        sem_i, sem_r, sem_s = sems[:2], sems[2:4], sems[4:]
        wid = lax.axis_index("s") * _NUM_CORES + lax.axis_index("c")
        base_b = wid * b_per_w
        base_r = base_b * fields

        def issue_idx(g, b):
            pltpu.async_copy(
                idx_hbm.at[pl.ds(base_r + g * rows, rows)],
                idx_v.at[b, pl.ds(0, rows)], sem_i[b])

        def drain_idx(b):
            pltpu.make_async_copy(
                idx_hbm.at[pl.ds(0, rows)], idx_v.at[b, pl.ds(0, rows)],
                sem_i[b]).wait()

        def fire_rows(b):
            for j in range(rows):
                if j % 16 == 0:
                    vec = idx_v[b, pl.ds(j, 16)]
                tid = vec[j % 16]
                pltpu.async_copy(
                    table_hbm.at[tid], stage_v.at[b, j // fields, j % fields],
                    sem_r[b])

        def drain_rows(b):
            pltpu.make_async_copy(
                out_hbm.at[pl.ds(0, _GB)], stage_v.at[b], sem_r[b]).wait()

        def issue_store(g, b):
            pltpu.async_copy(
                stage_v.at[b], out_hbm.at[pl.ds(base_b + g * _GB, _GB)],
                sem_s[b])

        def drain_store(b):
            pltpu.make_async_copy(
                stage_v.at[b], out_hbm.at[pl.ds(0, _GB)], sem_s[b]).wait()

        issue_idx(0, 0)
        issue_idx(1, 1)
        drain_idx(0)
        fire_rows(0)
        issue_idx(2, 0)
        drain_idx(1)
        fire_rows(1)
        issue_idx(3, 1)
        drain_rows(0)
        issue_store(0, 0)

        def body(p, carry):
            for b in (0, 1):
                g = 2 * p + b
                drain_idx(b)
                drain_store(b)
                fire_rows(b)
                issue_idx(g + 2, b)
                o = 1 - b
                drain_rows(o)
                issue_store(g - 1, o)
            return carry

        lax.fori_loop(1, n_pairs - 1, body, 0)

        for b in (0, 1):
            g = n_groups - 2 + b
            drain_idx(b)
            drain_store(b)
            fire_rows(b)
            o = 1 - b
            drain_rows(o)
            issue_store(g - 1, o)
        drain_rows(1)
        issue_store(n_groups - 1, 1)
        drain_store(0)
        drain_store(1)

    return k


def kernel(token_ids, E):
    batch, fields = token_ids.shape
    D = E.shape[1]
    flat = token_ids.reshape(batch * fields).astype(jnp.int32)
    return _make_gather(batch, fields, D)(flat, E)


# speedup vs baseline: 1.6390x; 1.2132x over previous
import functools

import jax
import jax.numpy as jnp
from jax import lax
from jax.experimental import pallas as pl
from jax.experimental.pallas import tpu as pltpu
from jax.experimental.pallas import tpu_sc as plsc

_NUM_CORES = 2
_NUM_SUBCORES = 16
_NUM_WORKERS = _NUM_CORES * _NUM_SUBCORES
_GB = 4


@functools.lru_cache(maxsize=None)
def _make_gather(batch, fields, D):
    b_per_w = batch // _NUM_WORKERS
    n_groups = b_per_w // _GB
    n_pairs = n_groups // 2
    rows = _GB * fields
    fp2 = fields // 2
    assert b_per_w * _NUM_WORKERS == batch and n_groups % 2 == 0 and n_groups >= 6
    assert rows % 8 == 0 and fields % 2 == 0
    mesh = plsc.VectorSubcoreMesh(core_axis_name="c", subcore_axis_name="s")

    @functools.partial(
        pl.kernel,
        mesh=mesh,
        out_type=jax.ShapeDtypeStruct((batch, fp2, 2 * D), jnp.float32),
        scratch_types=[
            pltpu.VMEM((2, rows + 16), jnp.int32),
            pltpu.VMEM((2, _GB, fp2, 2 * D), jnp.float32),
        ]
        + [pltpu.SemaphoreType.DMA] * 6,
    )
    def k(idx_hbm, table_hbm, out_hbm, idx_v, stage_v, *sems):
        sem_i, sem_r, sem_s = sems[:2], sems[2:4], sems[4:]
        wid = lax.axis_index("s") * _NUM_CORES + lax.axis_index("c")
        base_b = wid * b_per_w
        base_r = base_b * fields

        def issue_idx(g, b):
            pltpu.async_copy(
                idx_hbm.at[pl.ds(base_r + g * rows, rows)],
                idx_v.at[b, pl.ds(0, rows)], sem_i[b])

        def drain_idx(b):
            pltpu.make_async_copy(
                idx_hbm.at[pl.ds(0, rows)], idx_v.at[b, pl.ds(0, rows)],
                sem_i[b]).wait()

        def fire_rows(b):
            for j in range(rows):
                if j % 16 == 0:
                    vec = idx_v[b, pl.ds(j, 16)]
                tid = vec[j % 16]
                f = j % fields
                pltpu.async_copy(
                    table_hbm.at[tid],
                    stage_v.at[b, j // fields, f // 2, pl.ds((f % 2) * D, D)],
                    sem_r[b])

        def drain_rows(b):
            pltpu.make_async_copy(
                out_hbm.at[pl.ds(0, _GB)], stage_v.at[b], sem_r[b]).wait()

        def issue_store(g, b):
            pltpu.async_copy(
                stage_v.at[b], out_hbm.at[pl.ds(base_b + g * _GB, _GB)],
                sem_s[b])

        def drain_store(b):
            pltpu.make_async_copy(
                stage_v.at[b], out_hbm.at[pl.ds(0, _GB)], sem_s[b]).wait()

        issue_idx(0, 0)
        issue_idx(1, 1)
        drain_idx(0)
        fire_rows(0)
        issue_idx(2, 0)
        drain_idx(1)
        fire_rows(1)
        issue_idx(3, 1)
        drain_rows(0)
        issue_store(0, 0)

        def body(p, carry):
            for b in (0, 1):
                g = 2 * p + b
                drain_idx(b)
                drain_store(b)
                fire_rows(b)
                issue_idx(g + 2, b)
                o = 1 - b
                drain_rows(o)
                issue_store(g - 1, o)
            return carry

        lax.fori_loop(1, n_pairs - 1, body, 0)

        for b in (0, 1):
            g = n_groups - 2 + b
            drain_idx(b)
            drain_store(b)
            fire_rows(b)
            o = 1 - b
            drain_rows(o)
            issue_store(g - 1, o)
        drain_rows(1)
        issue_store(n_groups - 1, 1)
        drain_store(0)
        drain_store(1)

    return k


def kernel(token_ids, E):
    batch, fields = token_ids.shape
    D = E.shape[1]
    flat = token_ids.reshape(batch * fields).astype(jnp.int32)
    out = _make_gather(batch, fields, D)(flat, E)
    return out.reshape(batch, fields, D)


# speedup vs baseline: 1.6774x; 1.0234x over previous
import functools

import jax
import jax.numpy as jnp
from jax import lax
from jax.experimental import pallas as pl
from jax.experimental.pallas import tpu as pltpu
from jax.experimental.pallas import tpu_sc as plsc

_NUM_CORES = 2
_NUM_SUBCORES = 16
_NUM_WORKERS = _NUM_CORES * _NUM_SUBCORES
_GB = 4


@functools.lru_cache(maxsize=None)
def _make_gather(batch, fields, D):
    b_per_w = batch // _NUM_WORKERS
    n_groups = b_per_w // _GB
    n_pairs = n_groups // 2
    rows = _GB * fields
    fp2 = fields // 2
    assert b_per_w * _NUM_WORKERS == batch and n_groups % 2 == 0 and n_groups >= 6
    assert rows % 8 == 0 and fields % 2 == 0
    mesh = plsc.VectorSubcoreMesh(core_axis_name="c", subcore_axis_name="s")

    @functools.partial(
        pl.kernel,
        mesh=mesh,
        out_type=jax.ShapeDtypeStruct((batch, fields * D), jnp.float32),
        scratch_types=[
            pltpu.VMEM((2, rows + 16), jnp.int32),
            pltpu.VMEM((2, _GB, fields * D), jnp.float32),
        ]
        + [pltpu.SemaphoreType.DMA] * 6,
    )
    def k(idx_hbm, table_hbm, out_hbm, idx_v, stage_v, *sems):
        sem_i, sem_r, sem_s = sems[:2], sems[2:4], sems[4:]
        wid = lax.axis_index("s") * _NUM_CORES + lax.axis_index("c")
        base_b = wid * b_per_w
        base_r = base_b * fields

        def issue_idx(g, b):
            pltpu.async_copy(
                idx_hbm.at[pl.ds(base_r + g * rows, rows)],
                idx_v.at[b, pl.ds(0, rows)], sem_i[b])

        def drain_idx(b):
            pltpu.make_async_copy(
                idx_hbm.at[pl.ds(0, rows)], idx_v.at[b, pl.ds(0, rows)],
                sem_i[b]).wait()

        def fire_rows(b):
            for j in range(rows):
                if j % 16 == 0:
                    vec = idx_v[b, pl.ds(j, 16)]
                tid = vec[j % 16]
                f = j % fields
                pltpu.async_copy(
                    table_hbm.at[tid],
                    stage_v.at[b, j // fields, pl.ds(f * D, D)],
                    sem_r[b])

        def drain_rows(b):
            pltpu.make_async_copy(
                out_hbm.at[pl.ds(0, _GB)], stage_v.at[b], sem_r[b]).wait()

        def issue_store(g, b):
            pltpu.async_copy(
                stage_v.at[b], out_hbm.at[pl.ds(base_b + g * _GB, _GB)],
                sem_s[b])

        def drain_store(b):
            pltpu.make_async_copy(
                stage_v.at[b], out_hbm.at[pl.ds(0, _GB)], sem_s[b]).wait()

        issue_idx(0, 0)
        issue_idx(1, 1)
        drain_idx(0)
        fire_rows(0)
        issue_idx(2, 0)
        drain_idx(1)
        fire_rows(1)
        issue_idx(3, 1)
        drain_rows(0)
        issue_store(0, 0)

        def body(p, carry):
            for b in (0, 1):
                g = 2 * p + b
                drain_idx(b)
                drain_store(b)
                fire_rows(b)
                issue_idx(g + 2, b)
                o = 1 - b
                drain_rows(o)
                issue_store(g - 1, o)
            return carry

        lax.fori_loop(1, n_pairs - 1, body, 0)

        for b in (0, 1):
            g = n_groups - 2 + b
            drain_idx(b)
            drain_store(b)
            fire_rows(b)
            o = 1 - b
            drain_rows(o)
            issue_store(g - 1, o)
        drain_rows(1)
        issue_store(n_groups - 1, 1)
        drain_store(0)
        drain_store(1)

    return k


def kernel(token_ids, E):
    batch, fields = token_ids.shape
    D = E.shape[1]
    flat = token_ids.reshape(batch * fields).astype(jnp.int32)
    out = _make_gather(batch, fields, D)(flat, E)
    return out.reshape(batch, fields, D)
